# baseline (speedup 1.0000x reference)
import functools

import jax
import jax.numpy as jnp
from jax import lax
from jax.experimental import pallas as pl
from jax.experimental.pallas import tpu as pltpu
from jax.experimental.pallas import tpu_sc as plsc

_NC = 2
_NS = 16
_NW = _NC * _NS
_L = 16
_B = 128
_CHUNK = 1024
_UNROLL = 4


def _rsqrt(x):
    i = plsc.bitcast(x, jnp.int32)
    y = plsc.bitcast(jnp.int32(0x5F3759DF) - (i >> 1), jnp.float32)
    for _ in range(3):
        y = y * (1.5 - 0.5 * x * y * y)
    return y


def _inv_norm4(w, x, y, z):
    n2 = (w * w + x * x) + (y * y + z * z)
    nrm = n2 * _rsqrt(jnp.maximum(n2, 1e-30))
    return 1.0 / (nrm + 1e-8)


def _rotmat(w, x, y, z):
    n2 = (w * w + x * x) + (y * y + z * z)
    s = 2.0 / (n2 + 1e-30)
    xs, ys, zs = x * s, y * s, z * s
    wx, wy, wz = w * xs, w * ys, w * zs
    xx, xy, xz = x * xs, x * ys, x * zs
    yy, yz, zz = y * ys, y * zs, z * zs
    return ((1.0 - (yy + zz), xy - wz, xz + wy),
            (xy + wz, 1.0 - (xx + zz), yz - wx),
            (xz - wy, yz + wx, 1.0 - (xx + yy)))


@functools.lru_cache(maxsize=None)
def _build(N, M):
    assert N % (_NW * _CHUNK) == 0 and M % (_NS * _L) == 0
    assert _CHUNK % _B == 0
    G = N // _NW
    NCH = G // _CHUNK
    assert NCH % 2 == 0
    GROUPS = _CHUNK // _L
    MSL = M // _NS

    mesh = plsc.VectorSubcoreMesh(core_axis_name="c", subcore_axis_name="s")
    f32 = jnp.float32

    def in_set():
        return (pltpu.VMEM((7 * _CHUNK,), f32),
                pltpu.VMEM((4 * _CHUNK,), f32),
                pltpu.VMEM((8 * _CHUNK,), f32),
                pltpu.VMEM((8 * _CHUNK,), jnp.int32))

    @functools.partial(
        pl.kernel,
        out_type=(
            jax.ShapeDtypeStruct((3 * N,), f32),
            jax.ShapeDtypeStruct((9 * N,), f32),
            jax.ShapeDtypeStruct((3 * N,), f32),
            jax.ShapeDtypeStruct((N,), f32),
        ),
        mesh=mesh,
        compiler_params=pltpu.CompilerParams(
            needs_layout_passes=False,
            use_tc_tiling_on_sc=False,
        ),
        scratch_types=(
            pltpu.VMEM_SHARED((7 * M,), f32),
            [pltpu.VMEM((M,), f32) for _ in range(7)],
            pltpu.VMEM((14 * MSL,), f32),
            pltpu.VMEM((7 * MSL,), f32),
            in_set(),
            in_set(),
            pltpu.VMEM((16 * _CHUNK,), f32),
            pltpu.SemaphoreType.DMA,
            pltpu.SemaphoreType.DMA,
            pltpu.SemaphoreType.DMA,
        ),
    )
    def skin(qx_h, qq_h, sc_h, op_h, ind_h, w_h,
             nrx_h, nrq_h, ntx_h, ntq_h,
             mu_h, fr_h, s_h, o_h,
             shared, tb, nin, tsl, setA, setB, cfout, semA, semB, osem):
        ci = lax.axis_index("c")
        si = lax.axis_index("s")
        wid = si * _NC + ci

        iota = jnp.arange(_L, dtype=jnp.int32)

        nb = si * MSL
        handles = []
        for cc in range(3):
            handles.append(pltpu.async_copy(
                nrx_h.at[pl.ds(cc * M + nb, MSL)],
                nin.at[pl.ds(cc * MSL, MSL)], semA))
            handles.append(pltpu.async_copy(
                ntx_h.at[pl.ds(cc * M + nb, MSL)],
                nin.at[pl.ds((3 + cc) * MSL, MSL)], semA))
        handles.append(pltpu.async_copy(
            nrq_h.at[pl.ds(nb * 4, 4 * MSL)],
            nin.at[pl.ds(6 * MSL, 4 * MSL)], semA))
        handles.append(pltpu.async_copy(
            ntq_h.at[pl.ds(nb * 4, 4 * MSL)],
            nin.at[pl.ds(10 * MSL, 4 * MSL)], semA))
        for h in handles:
            h.wait()

        def node_group(g, carry):
            def ld(row):
                return plsc.load_gather(nin, [iota + (row * MSL + g * _L)])
            blk = g // (_B // _L)
            off = (g % (_B // _L)) * _L
            qb_ = blk * (4 * _B) + off

            def ldq(qoff, cc):
                return plsc.load_gather(
                    nin, [iota + (qoff + qb_ + cc * _B)])
            rv = [ld(0), ld(1), ld(2)]
            rq = [ldq(6 * MSL, cc) for cc in range(4)]
            tv = [ld(3), ld(4), ld(5)]
            tq = [ldq(10 * MSL, cc) for cc in range(4)]
            rinv = _inv_norm4(*rq)
            tinv = _inv_norm4(*tq)
            aw, ax, ay, az = (q * tinv for q in tq)
            bw = rq[0] * rinv
            bx = -rq[1] * rinv
            by = -rq[2] * rinv
            bz = -rq[3] * rinv
            dw = aw * bw - ax * bx - ay * by - az * bz
            dx = aw * bx + ax * bw + ay * bz - az * by
            dy = aw * by - ax * bz + ay * bw + az * bx
            dz = aw * bz + ax * by - ay * bx + az * bw
            R = _rotmat(dw, dx, dy, dz)
            t = [tv[r] - (R[r][0] * rv[0] + R[r][1] * rv[1] + R[r][2] * rv[2])
                 for r in range(3)]
            base = g * _L
            for row, val in enumerate((dw, dx, dy, dz, t[0], t[1], t[2])):
                plsc.store_scatter(tsl, [iota + (row * MSL + base)], val)
            return carry

        lax.fori_loop(0, MSL // _L, node_group, 0)
        for comp in range(7):
            pltpu.sync_copy(tsl.at[pl.ds(comp * MSL, MSL)],
                            shared.at[pl.ds(comp * M + si * MSL, MSL)])
        plsc.subcore_barrier()
        for comp in range(7):
            pltpu.sync_copy(shared.at[pl.ds(comp * M, M)], tb[comp])

        g0 = wid * G

        def in_copies(bufs, sem, b):
            cfin, cqq, cw, cind = bufs
            cps = []
            for cc in range(3):
                cps.append((qx_h.at[pl.ds(cc * N + b, _CHUNK)],
                            cfin.at[pl.ds(cc * _CHUNK, _CHUNK)], sem))
                cps.append((sc_h.at[pl.ds(cc * N + b, _CHUNK)],
                            cfin.at[pl.ds((3 + cc) * _CHUNK, _CHUNK)], sem))
            cps.append((op_h.at[pl.ds(b, _CHUNK)],
                        cfin.at[pl.ds(6 * _CHUNK, _CHUNK)], sem))
            cps.append((qq_h.at[pl.ds(b * 4, 4 * _CHUNK)], cqq, sem))
            cps.append((w_h.at[pl.ds(b * 8, 8 * _CHUNK)], cw, sem))
            cps.append((ind_h.at[pl.ds(b * 8, 8 * _CHUNK)], cind, sem))
            return cps

        def issue(copies):
            for s, d, sem in copies:
                pltpu.async_copy(s, d, sem)

        def drain(copies):
            for s, d, sem in copies:
                pltpu.make_async_copy(s, d, sem).wait()

        def do_group(bufs, g):
            cfin, cqq, cw, cind = bufs
            base = g * _L
            blk = g // (_B // _L)
            off = (g % (_B // _L)) * _L
            qbase = blk * (4 * _B) + off
            wbase = blk * (8 * _B) + off

            ks = [cind[pl.ds(wbase + k * _B, _L)] for k in range(8)]
            ws = [cw[pl.ds(wbase + k * _B, _L)] for k in range(8)]
            wsum = (((ws[0] + ws[1]) + (ws[2] + ws[3]))
                    + ((ws[4] + ws[5]) + (ws[6] + ws[7])))
            winv = 1.0 / (wsum + 1e-8)

            q0 = [plsc.load_gather(tb[cc], [ks[0]]) for cc in range(4)]
            wk = [ws[0]]
            for k in range(1, 8):
                qk = [plsc.load_gather(tb[cc], [ks[k]]) for cc in range(4)]
                d = ((q0[0] * qk[0] + q0[1] * qk[1])
                     + (q0[2] * qk[2] + q0[3] * qk[3]))
                wk.append(jnp.where(d < 0, -ws[k], ws[k]))
                if k == 1:
                    aq = [wk[1] * q for q in qk]
                else:
                    aq = [aq[cc] + wk[k] * qk[cc] for cc in range(4)]
            aq = [aq[cc] + wk[0] * q0[cc] for cc in range(4)]
            at = None
            for k in range(8):
                tk = [plsc.load_gather(tb[4 + cc], [ks[k]])
                      for cc in range(3)]
                if at is None:
                    at = [ws[0] * t for t in tk]
                else:
                    at = [at[cc] + ws[k] * tk[cc] for cc in range(3)]

            tb_ = [a * winv for a in at]
            Rb = _rotmat(aq[0], aq[1], aq[2], aq[3])

            def ldf(row):
                return cfin[pl.ds(row * _CHUNK + base, _L)]

            def stf(row, val):
                cfout[pl.ds(row * _CHUNK + base, _L)] = val

            v = [ldf(0), ldf(1), ldf(2)]
            for r in range(3):
                stf(r, (Rb[r][0] * v[0] + Rb[r][1] * v[1])
                    + (Rb[r][2] * v[2] + tb_[r]))

            qr = [cqq[pl.ds(qbase + cc * _B, _L)] for cc in range(4)]
            mw = aq[0] * qr[0] - aq[1] * qr[1] - aq[2] * qr[2] - aq[3] * qr[3]
            mx = aq[0] * qr[1] + aq[1] * qr[0] + aq[2] * qr[3] - aq[3] * qr[2]
            my = aq[0] * qr[2] - aq[1] * qr[3] + aq[2] * qr[0] + aq[3] * qr[1]
            mz = aq[0] * qr[3] + aq[1] * qr[2] - aq[2] * qr[1] + aq[3] * qr[0]
            Rm = _rotmat(mw, mx, my, mz)
            for r in range(3):
                for col in range(3):
                    stf(3 + 3 * r + col, Rm[r][col])

            for cc in range(3):
                stf(12 + cc, jnp.exp(ldf(3 + cc)))
            stf(15, 1.0 / (1.0 + jnp.exp(-ldf(6))))

        def compute_chunk(bufs):
            def group_fn(j, carry2):
                for u in range(_UNROLL):
                    do_group(bufs, j * _UNROLL + u)
                return carry2
            lax.fori_loop(0, GROUPS // _UNROLL, group_fn, 0)

        def out_copies(b):
            cps = []
            for r in range(3):
                cps.append((cfout.at[pl.ds(r * _CHUNK, _CHUNK)],
                            mu_h.at[pl.ds(r * N + b, _CHUNK)], osem))
            for r in range(9):
                cps.append((cfout.at[pl.ds((3 + r) * _CHUNK, _CHUNK)],
                            fr_h.at[pl.ds(r * N + b, _CHUNK)], osem))
            for r in range(3):
                cps.append((cfout.at[pl.ds((12 + r) * _CHUNK, _CHUNK)],
                            s_h.at[pl.ds(r * N + b, _CHUNK)], osem))
            cps.append((cfout.at[pl.ds(15 * _CHUNK, _CHUNK)],
                        o_h.at[pl.ds(b, _CHUNK)], osem))
            return cps

        issue(in_copies(setA, semA, g0))

        def body(i, carry):
            b0 = g0 + (2 * i) * _CHUNK
            b1 = b0 + _CHUNK
            issue(in_copies(setB, semB, b1))
            drain(in_copies(setA, semA, b0))
            compute_chunk(setA)
            oc = out_copies(b0)
            issue(oc)
            drain(oc)

            @pl.when(i + 1 < NCH // 2)
            def _prefetch():
                issue(in_copies(setA, semA, b1 + _CHUNK))

            drain(in_copies(setB, semB, b1))
            compute_chunk(setB)
            oc = out_copies(b1)
            issue(oc)
            drain(oc)
            return carry

        lax.fori_loop(0, NCH // 2, body, 0)

    return skin


def kernel(query_xyz, query_quats, scales, opacities, sph, sk_ind, sk_w,
           node_ref_xyz, node_ref_quat, node_tgt_xyz, node_tgt_quat):
    N = query_xyz.shape[0]
    M = node_ref_xyz.shape[0]
    assert sk_ind.shape[1] == 8

    def blocks(a):
        rows, C = a.shape
        return a.T.reshape(C, rows // _B, _B).transpose(1, 0, 2).reshape(-1)

    mu_t, fr_t, s_t, o = _build(N, M)(
        query_xyz.T.reshape(-1),
        blocks(query_quats),
        scales.T.reshape(-1),
        opacities,
        blocks(sk_ind.astype(jnp.int32)),
        blocks(sk_w),
        node_ref_xyz.T.reshape(-1),
        blocks(node_ref_quat),
        node_tgt_xyz.T.reshape(-1),
        blocks(node_tgt_quat),
    )
    mu = mu_t.reshape(3, N).T
    fr = fr_t.reshape(3, 3, N).transpose(2, 0, 1)
    s = s_t.reshape(3, N).T
    return (mu, fr, s, o, sph)


# speedup vs baseline: 1.1364x; 1.1364x over previous
import functools

import jax
import jax.numpy as jnp
from jax import lax
from jax.experimental import pallas as pl
from jax.experimental.pallas import tpu as pltpu
from jax.experimental.pallas import tpu_sc as plsc

_NC = 2
_NS = 16
_NW = _NC * _NS
_L = 16
_B = 128
_CHUNK = 1024
_UNROLL = 2


def _rsqrt(x):
    i = plsc.bitcast(x, jnp.int32)
    y = plsc.bitcast(jnp.int32(0x5F3759DF) - (i >> 1), jnp.float32)
    for _ in range(3):
        y = y * (1.5 - 0.5 * x * y * y)
    return y


def _inv_norm4(w, x, y, z):
    n2 = (w * w + x * x) + (y * y + z * z)
    nrm = n2 * _rsqrt(jnp.maximum(n2, 1e-30))
    return 1.0 / (nrm + 1e-8)


def _rotmat(w, x, y, z):
    n2 = (w * w + x * x) + (y * y + z * z)
    s = 2.0 / (n2 + 1e-30)
    xs, ys, zs = x * s, y * s, z * s
    wx, wy, wz = w * xs, w * ys, w * zs
    xx, xy, xz = x * xs, x * ys, x * zs
    yy, yz, zz = y * ys, y * zs, z * zs
    return ((1.0 - (yy + zz), xy - wz, xz + wy),
            (xy + wz, 1.0 - (xx + zz), yz - wx),
            (xz - wy, yz + wx, 1.0 - (xx + yy)))


@functools.lru_cache(maxsize=None)
def _build(N, M):
    assert N % (_NW * _CHUNK) == 0 and M % (_NS * _L) == 0
    assert _CHUNK % _B == 0
    G = N // _NW
    NCH = G // _CHUNK
    assert NCH % 2 == 0
    GROUPS = _CHUNK // _L
    MSL = M // _NS

    mesh = plsc.VectorSubcoreMesh(core_axis_name="c", subcore_axis_name="s")
    f32 = jnp.float32

    def in_set():
        return (pltpu.VMEM((7 * _CHUNK,), f32),
                pltpu.VMEM((4 * _CHUNK,), f32),
                pltpu.VMEM((8 * _CHUNK,), f32),
                pltpu.VMEM((8 * _CHUNK,), jnp.int32))

    @functools.partial(
        pl.kernel,
        out_type=(
            jax.ShapeDtypeStruct((4 * N,), f32),
            jax.ShapeDtypeStruct((12 * N,), f32),
            jax.ShapeDtypeStruct((4 * N,), f32),
            jax.ShapeDtypeStruct((N,), f32),
        ),
        mesh=mesh,
        compiler_params=pltpu.CompilerParams(
            needs_layout_passes=False,
            use_tc_tiling_on_sc=False,
        ),
        scratch_types=(
            pltpu.VMEM_SHARED((7 * M,), f32),
            [pltpu.VMEM((M,), f32) for _ in range(7)],
            pltpu.VMEM((14 * MSL,), f32),
            pltpu.VMEM((7 * MSL,), f32),
            in_set(),
            in_set(),
            pltpu.VMEM((21 * _CHUNK,), f32),
            pltpu.SemaphoreType.DMA,
            pltpu.SemaphoreType.DMA,
            pltpu.SemaphoreType.DMA,
        ),
    )
    def skin(qx_h, qq_h, sc_h, op_h, ind_h, w_h,
             nrx_h, nrq_h, ntx_h, ntq_h,
             mu_h, fr_h, s_h, o_h,
             shared, tb, nin, tsl, setA, setB, cfout, semA, semB, osem):
        ci = lax.axis_index("c")
        si = lax.axis_index("s")
        wid = si * _NC + ci

        iota = jnp.arange(_L, dtype=jnp.int32)

        nb = si * MSL
        handles = []
        for cc in range(3):
            handles.append(pltpu.async_copy(
                nrx_h.at[pl.ds(cc * M + nb, MSL)],
                nin.at[pl.ds(cc * MSL, MSL)], semA))
            handles.append(pltpu.async_copy(
                ntx_h.at[pl.ds(cc * M + nb, MSL)],
                nin.at[pl.ds((3 + cc) * MSL, MSL)], semA))
        handles.append(pltpu.async_copy(
            nrq_h.at[pl.ds(nb * 4, 4 * MSL)],
            nin.at[pl.ds(6 * MSL, 4 * MSL)], semA))
        handles.append(pltpu.async_copy(
            ntq_h.at[pl.ds(nb * 4, 4 * MSL)],
            nin.at[pl.ds(10 * MSL, 4 * MSL)], semA))
        for h in handles:
            h.wait()

        def node_group(g, carry):
            def ld(row):
                return plsc.load_gather(nin, [iota + (row * MSL + g * _L)])
            blk = g // (_B // _L)
            off = (g % (_B // _L)) * _L
            qb_ = blk * (4 * _B) + off

            def ldq(qoff, cc):
                return plsc.load_gather(
                    nin, [iota + (qoff + qb_ + cc * _B)])
            rv = [ld(0), ld(1), ld(2)]
            rq = [ldq(6 * MSL, cc) for cc in range(4)]
            tv = [ld(3), ld(4), ld(5)]
            tq = [ldq(10 * MSL, cc) for cc in range(4)]
            rinv = _inv_norm4(*rq)
            tinv = _inv_norm4(*tq)
            aw, ax, ay, az = (q * tinv for q in tq)
            bw = rq[0] * rinv
            bx = -rq[1] * rinv
            by = -rq[2] * rinv
            bz = -rq[3] * rinv
            dw = aw * bw - ax * bx - ay * by - az * bz
            dx = aw * bx + ax * bw + ay * bz - az * by
            dy = aw * by - ax * bz + ay * bw + az * bx
            dz = aw * bz + ax * by - ay * bx + az * bw
            R = _rotmat(dw, dx, dy, dz)
            t = [tv[r] - (R[r][0] * rv[0] + R[r][1] * rv[1] + R[r][2] * rv[2])
                 for r in range(3)]
            base = g * _L
            for row, val in enumerate((dw, dx, dy, dz, t[0], t[1], t[2])):
                plsc.store_scatter(tsl, [iota + (row * MSL + base)], val)
            return carry

        lax.fori_loop(0, MSL // _L, node_group, 0)
        for comp in range(7):
            pltpu.sync_copy(tsl.at[pl.ds(comp * MSL, MSL)],
                            shared.at[pl.ds(comp * M + si * MSL, MSL)])
        plsc.subcore_barrier()
        for comp in range(7):
            pltpu.sync_copy(shared.at[pl.ds(comp * M, M)], tb[comp])

        g0 = wid * G

        def in_copies(bufs, sem, b):
            cfin, cqq, cw, cind = bufs
            cps = []
            for cc in range(3):
                cps.append((qx_h.at[pl.ds(cc * N + b, _CHUNK)],
                            cfin.at[pl.ds(cc * _CHUNK, _CHUNK)], sem))
                cps.append((sc_h.at[pl.ds(cc * N + b, _CHUNK)],
                            cfin.at[pl.ds((3 + cc) * _CHUNK, _CHUNK)], sem))
            cps.append((op_h.at[pl.ds(b, _CHUNK)],
                        cfin.at[pl.ds(6 * _CHUNK, _CHUNK)], sem))
            cps.append((qq_h.at[pl.ds(b * 4, 4 * _CHUNK)], cqq, sem))
            cps.append((w_h.at[pl.ds(b * 8, 8 * _CHUNK)], cw, sem))
            cps.append((ind_h.at[pl.ds(b * 8, 8 * _CHUNK)], cind, sem))
            return cps

        def issue(copies):
            for s, d, sem in copies:
                pltpu.async_copy(s, d, sem)

        def drain(copies):
            for s, d, sem in copies:
                pltpu.make_async_copy(s, d, sem).wait()

        def do_group(bufs, g):
            cfin, cqq, cw, cind = bufs
            base = g * _L
            blk = g // (_B // _L)
            off = (g % (_B // _L)) * _L
            qbase = blk * (4 * _B) + off
            wbase = blk * (8 * _B) + off

            ks = [cind[pl.ds(wbase + k * _B, _L)] for k in range(8)]
            ws = [cw[pl.ds(wbase + k * _B, _L)] for k in range(8)]
            wsum = (((ws[0] + ws[1]) + (ws[2] + ws[3]))
                    + ((ws[4] + ws[5]) + (ws[6] + ws[7])))
            winv = 1.0 / (wsum + 1e-8)

            q0 = [plsc.load_gather(tb[cc], [ks[0]]) for cc in range(4)]
            wk = [ws[0]]
            for k in range(1, 8):
                qk = [plsc.load_gather(tb[cc], [ks[k]]) for cc in range(4)]
                d = ((q0[0] * qk[0] + q0[1] * qk[1])
                     + (q0[2] * qk[2] + q0[3] * qk[3]))
                wk.append(jnp.where(d < 0, -ws[k], ws[k]))
                if k == 1:
                    aq = [wk[1] * q for q in qk]
                else:
                    aq = [aq[cc] + wk[k] * qk[cc] for cc in range(4)]
            aq = [aq[cc] + wk[0] * q0[cc] for cc in range(4)]
            at = None
            for k in range(8):
                tk = [plsc.load_gather(tb[4 + cc], [ks[k]])
                      for cc in range(3)]
                if at is None:
                    at = [ws[0] * t for t in tk]
                else:
                    at = [at[cc] + ws[k] * tk[cc] for cc in range(3)]

            tb_ = [a * winv for a in at]
            Rb = _rotmat(aq[0], aq[1], aq[2], aq[3])

            def ldf(row):
                return cfin[pl.ds(row * _CHUNK + base, _L)]

            mub = blk * (4 * _B) + off
            v = [ldf(0), ldf(1), ldf(2)]
            for r in range(3):
                cfout[pl.ds(mub + r * _B, _L)] = (
                    (Rb[r][0] * v[0] + Rb[r][1] * v[1])
                    + (Rb[r][2] * v[2] + tb_[r]))

            qr = [cqq[pl.ds(qbase + cc * _B, _L)] for cc in range(4)]
            mw = aq[0] * qr[0] - aq[1] * qr[1] - aq[2] * qr[2] - aq[3] * qr[3]
            mx = aq[0] * qr[1] + aq[1] * qr[0] + aq[2] * qr[3] - aq[3] * qr[2]
            my = aq[0] * qr[2] - aq[1] * qr[3] + aq[2] * qr[0] + aq[3] * qr[1]
            mz = aq[0] * qr[3] + aq[1] * qr[2] - aq[2] * qr[1] + aq[3] * qr[0]
            Rm = _rotmat(mw, mx, my, mz)
            for r in range(3):
                frb = (4 + 4 * r) * _CHUNK + mub
                for col in range(3):
                    cfout[pl.ds(frb + col * _B, _L)] = Rm[r][col]

            sb = 16 * _CHUNK + mub
            for cc in range(3):
                cfout[pl.ds(sb + cc * _B, _L)] = jnp.exp(ldf(3 + cc))
            cfout[pl.ds(20 * _CHUNK + base, _L)] = (
                1.0 / (1.0 + jnp.exp(-ldf(6))))

        def compute_chunk(bufs):
            def group_fn(j, carry2):
                for u in range(_UNROLL):
                    do_group(bufs, j * _UNROLL + u)
                return carry2
            lax.fori_loop(0, GROUPS // _UNROLL, group_fn, 0)

        def out_copies(b):
            cps = [(cfout.at[pl.ds(0, 4 * _CHUNK)],
                    mu_h.at[pl.ds(b * 4, 4 * _CHUNK)], osem)]
            for r in range(3):
                cps.append((cfout.at[pl.ds((4 + 4 * r) * _CHUNK, 4 * _CHUNK)],
                            fr_h.at[pl.ds(r * (4 * N) + b * 4, 4 * _CHUNK)],
                            osem))
            cps.append((cfout.at[pl.ds(16 * _CHUNK, 4 * _CHUNK)],
                        s_h.at[pl.ds(b * 4, 4 * _CHUNK)], osem))
            cps.append((cfout.at[pl.ds(20 * _CHUNK, _CHUNK)],
                        o_h.at[pl.ds(b, _CHUNK)], osem))
            return cps

        issue(in_copies(setA, semA, g0))

        def body(i, carry):
            b0 = g0 + (2 * i) * _CHUNK
            b1 = b0 + _CHUNK
            issue(in_copies(setB, semB, b1))
            drain(in_copies(setA, semA, b0))
            compute_chunk(setA)
            oc = out_copies(b0)
            issue(oc)
            drain(oc)

            @pl.when(i + 1 < NCH // 2)
            def _prefetch():
                issue(in_copies(setA, semA, b1 + _CHUNK))

            drain(in_copies(setB, semB, b1))
            compute_chunk(setB)
            oc = out_copies(b1)
            issue(oc)
            drain(oc)
            return carry

        lax.fori_loop(0, NCH // 2, body, 0)

    return skin


def kernel(query_xyz, query_quats, scales, opacities, sph, sk_ind, sk_w,
           node_ref_xyz, node_ref_quat, node_tgt_xyz, node_tgt_quat):
    N = query_xyz.shape[0]
    M = node_ref_xyz.shape[0]
    assert sk_ind.shape[1] == 8

    def blocks(a):
        rows, C = a.shape
        return a.T.reshape(C, rows // _B, _B).transpose(1, 0, 2).reshape(-1)

    mu_t, fr_t, s_t, o = _build(N, M)(
        query_xyz.T.reshape(-1),
        blocks(query_quats),
        scales.T.reshape(-1),
        opacities,
        blocks(sk_ind.astype(jnp.int32)),
        blocks(sk_w),
        node_ref_xyz.T.reshape(-1),
        blocks(node_ref_quat),
        node_tgt_xyz.T.reshape(-1),
        blocks(node_tgt_quat),
    )
    mu = (mu_t.reshape(N // _B, 4, _B)[:, :3, :]
          .transpose(0, 2, 1).reshape(N, 3))
    fr = (fr_t.reshape(3, N // _B, 4, _B)[:, :, :3, :]
          .transpose(1, 3, 0, 2).reshape(N, 3, 3))
    s = (s_t.reshape(N // _B, 4, _B)[:, :3, :]
         .transpose(0, 2, 1).reshape(N, 3))
    return (mu, fr, s, o, sph)


# speedup vs baseline: 1.1559x; 1.0172x over previous
import functools

import jax
import jax.numpy as jnp
from jax import lax
from jax.experimental import pallas as pl
from jax.experimental.pallas import tpu as pltpu
from jax.experimental.pallas import tpu_sc as plsc

_NC = 2
_NS = 16
_NW = _NC * _NS
_L = 16
_B = 128
_CHUNK = 1024
_UNROLL = 2


def _rsqrt(x):
    i = plsc.bitcast(x, jnp.int32)
    y = plsc.bitcast(jnp.int32(0x5F3759DF) - (i >> 1), jnp.float32)
    for _ in range(3):
        y = y * (1.5 - 0.5 * x * y * y)
    return y


def _inv_norm4(w, x, y, z):
    n2 = (w * w + x * x) + (y * y + z * z)
    nrm = n2 * _rsqrt(jnp.maximum(n2, 1e-30))
    return 1.0 / (nrm + 1e-8)


def _rotmat(w, x, y, z):
    n2 = (w * w + x * x) + (y * y + z * z)
    s = 2.0 / (n2 + 1e-30)
    xs, ys, zs = x * s, y * s, z * s
    wx, wy, wz = w * xs, w * ys, w * zs
    xx, xy, xz = x * xs, x * ys, x * zs
    yy, yz, zz = y * ys, y * zs, z * zs
    return ((1.0 - (yy + zz), xy - wz, xz + wy),
            (xy + wz, 1.0 - (xx + zz), yz - wx),
            (xz - wy, yz + wx, 1.0 - (xx + yy)))


@functools.lru_cache(maxsize=None)
def _build(N, M):
    assert N % (_NW * _CHUNK) == 0 and M % (_NS * _L) == 0
    assert _CHUNK % _B == 0
    G = N // _NW
    NCH = G // _CHUNK
    assert NCH % 2 == 0
    GROUPS = _CHUNK // _L
    MSL = M // _NS

    mesh = plsc.VectorSubcoreMesh(core_axis_name="c", subcore_axis_name="s")
    f32 = jnp.float32

    def in_set():
        return (pltpu.VMEM((7 * _CHUNK,), f32),
                pltpu.VMEM((4 * _CHUNK,), f32),
                pltpu.VMEM((8 * _CHUNK,), f32),
                pltpu.VMEM((8 * _CHUNK,), jnp.int32))

    @functools.partial(
        pl.kernel,
        out_type=(
            jax.ShapeDtypeStruct((4 * N,), f32),
            jax.ShapeDtypeStruct((12 * N,), f32),
            jax.ShapeDtypeStruct((4 * N,), f32),
            jax.ShapeDtypeStruct((N,), f32),
        ),
        mesh=mesh,
        compiler_params=pltpu.CompilerParams(
            needs_layout_passes=False,
            use_tc_tiling_on_sc=False,
        ),
        scratch_types=(
            pltpu.VMEM_SHARED((7 * M,), f32),
            [pltpu.VMEM((M,), f32) for _ in range(7)],
            pltpu.VMEM((14 * MSL,), f32),
            pltpu.VMEM((7 * MSL,), f32),
            in_set(),
            in_set(),
            pltpu.VMEM((21 * _CHUNK,), f32),
            pltpu.SemaphoreType.DMA,
            pltpu.SemaphoreType.DMA,
            pltpu.SemaphoreType.DMA,
        ),
    )
    def skin(qx_h, qq_h, sc_h, op_h, ind_h, w_h,
             nrx_h, nrq_h, ntx_h, ntq_h,
             mu_h, fr_h, s_h, o_h,
             shared, tb, nin, tsl, setA, setB, cfout, semA, semB, osem):
        ci = lax.axis_index("c")
        si = lax.axis_index("s")
        wid = si * _NC + ci

        iota = jnp.arange(_L, dtype=jnp.int32)

        def in_copies(bufs, sem, b):
            cfin, cqq, cw, cind = bufs
            cps = []
            for cc in range(3):
                cps.append((qx_h.at[pl.ds(cc * N + b, _CHUNK)],
                            cfin.at[pl.ds(cc * _CHUNK, _CHUNK)], sem))
                cps.append((sc_h.at[pl.ds(cc * N + b, _CHUNK)],
                            cfin.at[pl.ds((3 + cc) * _CHUNK, _CHUNK)], sem))
            cps.append((op_h.at[pl.ds(b, _CHUNK)],
                        cfin.at[pl.ds(6 * _CHUNK, _CHUNK)], sem))
            cps.append((qq_h.at[pl.ds(b * 4, 4 * _CHUNK)], cqq, sem))
            cps.append((w_h.at[pl.ds(b * 8, 8 * _CHUNK)], cw, sem))
            cps.append((ind_h.at[pl.ds(b * 8, 8 * _CHUNK)], cind, sem))
            return cps

        def issue(copies):
            for s, d, sem in copies:
                pltpu.async_copy(s, d, sem)

        def drain(copies):
            for s, d, sem in copies:
                pltpu.make_async_copy(s, d, sem).wait()


        nb = si * MSL
        handles = []
        for cc in range(3):
            handles.append(pltpu.async_copy(
                nrx_h.at[pl.ds(cc * M + nb, MSL)],
                nin.at[pl.ds(cc * MSL, MSL)], semA))
            handles.append(pltpu.async_copy(
                ntx_h.at[pl.ds(cc * M + nb, MSL)],
                nin.at[pl.ds((3 + cc) * MSL, MSL)], semA))
        handles.append(pltpu.async_copy(
            nrq_h.at[pl.ds(nb * 4, 4 * MSL)],
            nin.at[pl.ds(6 * MSL, 4 * MSL)], semA))
        handles.append(pltpu.async_copy(
            ntq_h.at[pl.ds(nb * 4, 4 * MSL)],
            nin.at[pl.ds(10 * MSL, 4 * MSL)], semA))
        issue(in_copies(setA, semA, wid * G))
        for h in handles:
            h.wait()

        def node_group(g, carry):
            def ld(row):
                return plsc.load_gather(nin, [iota + (row * MSL + g * _L)])
            blk = g // (_B // _L)
            off = (g % (_B // _L)) * _L
            qb_ = blk * (4 * _B) + off

            def ldq(qoff, cc):
                return plsc.load_gather(
                    nin, [iota + (qoff + qb_ + cc * _B)])
            rv = [ld(0), ld(1), ld(2)]
            rq = [ldq(6 * MSL, cc) for cc in range(4)]
            tv = [ld(3), ld(4), ld(5)]
            tq = [ldq(10 * MSL, cc) for cc in range(4)]
            rinv = _inv_norm4(*rq)
            tinv = _inv_norm4(*tq)
            aw, ax, ay, az = (q * tinv for q in tq)
            bw = rq[0] * rinv
            bx = -rq[1] * rinv
            by = -rq[2] * rinv
            bz = -rq[3] * rinv
            dw = aw * bw - ax * bx - ay * by - az * bz
            dx = aw * bx + ax * bw + ay * bz - az * by
            dy = aw * by - ax * bz + ay * bw + az * bx
            dz = aw * bz + ax * by - ay * bx + az * bw
            R = _rotmat(dw, dx, dy, dz)
            t = [tv[r] - (R[r][0] * rv[0] + R[r][1] * rv[1] + R[r][2] * rv[2])
                 for r in range(3)]
            base = g * _L
            for row, val in enumerate((dw, dx, dy, dz, t[0], t[1], t[2])):
                plsc.store_scatter(tsl, [iota + (row * MSL + base)], val)
            return carry

        lax.fori_loop(0, MSL // _L, node_group, 0)
        for comp in range(7):
            pltpu.sync_copy(tsl.at[pl.ds(comp * MSL, MSL)],
                            shared.at[pl.ds(comp * M + si * MSL, MSL)])
        plsc.subcore_barrier()
        for comp in range(7):
            pltpu.sync_copy(shared.at[pl.ds(comp * M, M)], tb[comp])

        g0 = wid * G

        def do_group(bufs, g):
            cfin, cqq, cw, cind = bufs
            base = g * _L
            blk = g // (_B // _L)
            off = (g % (_B // _L)) * _L
            qbase = blk * (4 * _B) + off
            wbase = blk * (8 * _B) + off

            ks = [cind[pl.ds(wbase + k * _B, _L)] for k in range(8)]
            ws = [cw[pl.ds(wbase + k * _B, _L)] for k in range(8)]
            wsum = (((ws[0] + ws[1]) + (ws[2] + ws[3]))
                    + ((ws[4] + ws[5]) + (ws[6] + ws[7])))
            winv = 1.0 / (wsum + 1e-8)

            q0 = [plsc.load_gather(tb[cc], [ks[0]]) for cc in range(4)]
            wk = [ws[0]]
            for k in range(1, 8):
                qk = [plsc.load_gather(tb[cc], [ks[k]]) for cc in range(4)]
                d = ((q0[0] * qk[0] + q0[1] * qk[1])
                     + (q0[2] * qk[2] + q0[3] * qk[3]))
                wk.append(jnp.where(d < 0, -ws[k], ws[k]))
                if k == 1:
                    aq = [wk[1] * q for q in qk]
                else:
                    aq = [aq[cc] + wk[k] * qk[cc] for cc in range(4)]
            aq = [aq[cc] + wk[0] * q0[cc] for cc in range(4)]
            at = None
            for k in range(8):
                tk = [plsc.load_gather(tb[4 + cc], [ks[k]])
                      for cc in range(3)]
                if at is None:
                    at = [ws[0] * t for t in tk]
                else:
                    at = [at[cc] + ws[k] * tk[cc] for cc in range(3)]

            tb_ = [a * winv for a in at]
            Rb = _rotmat(aq[0], aq[1], aq[2], aq[3])

            def ldf(row):
                return cfin[pl.ds(row * _CHUNK + base, _L)]

            mub = blk * (4 * _B) + off
            v = [ldf(0), ldf(1), ldf(2)]
            for r in range(3):
                cfout[pl.ds(mub + r * _B, _L)] = (
                    (Rb[r][0] * v[0] + Rb[r][1] * v[1])
                    + (Rb[r][2] * v[2] + tb_[r]))

            qr = [cqq[pl.ds(qbase + cc * _B, _L)] for cc in range(4)]
            mw = aq[0] * qr[0] - aq[1] * qr[1] - aq[2] * qr[2] - aq[3] * qr[3]
            mx = aq[0] * qr[1] + aq[1] * qr[0] + aq[2] * qr[3] - aq[3] * qr[2]
            my = aq[0] * qr[2] - aq[1] * qr[3] + aq[2] * qr[0] + aq[3] * qr[1]
            mz = aq[0] * qr[3] + aq[1] * qr[2] - aq[2] * qr[1] + aq[3] * qr[0]
            Rm = _rotmat(mw, mx, my, mz)
            for r in range(3):
                frb = (4 + 4 * r) * _CHUNK + mub
                for col in range(3):
                    cfout[pl.ds(frb + col * _B, _L)] = Rm[r][col]

            sb = 16 * _CHUNK + mub
            for cc in range(3):
                cfout[pl.ds(sb + cc * _B, _L)] = jnp.exp(ldf(3 + cc))
            cfout[pl.ds(20 * _CHUNK + base, _L)] = (
                1.0 / (1.0 + jnp.exp(-ldf(6))))

        def compute_chunk(bufs):
            def group_fn(j, carry2):
                for u in range(_UNROLL):
                    do_group(bufs, j * _UNROLL + u)
                return carry2
            lax.fori_loop(0, GROUPS // _UNROLL, group_fn, 0)

        def out_copies(b):
            cps = [(cfout.at[pl.ds(0, 4 * _CHUNK)],
                    mu_h.at[pl.ds(b * 4, 4 * _CHUNK)], osem)]
            for r in range(3):
                cps.append((cfout.at[pl.ds((4 + 4 * r) * _CHUNK, 4 * _CHUNK)],
                            fr_h.at[pl.ds(r * (4 * N) + b * 4, 4 * _CHUNK)],
                            osem))
            cps.append((cfout.at[pl.ds(16 * _CHUNK, 4 * _CHUNK)],
                        s_h.at[pl.ds(b * 4, 4 * _CHUNK)], osem))
            cps.append((cfout.at[pl.ds(20 * _CHUNK, _CHUNK)],
                        o_h.at[pl.ds(b, _CHUNK)], osem))
            return cps

        def body(i, carry):
            b0 = g0 + (2 * i) * _CHUNK
            b1 = b0 + _CHUNK
            issue(in_copies(setB, semB, b1))
            drain(in_copies(setA, semA, b0))
            compute_chunk(setA)
            oc = out_copies(b0)
            issue(oc)
            drain(oc)

            @pl.when(i + 1 < NCH // 2)
            def _prefetch():
                issue(in_copies(setA, semA, b1 + _CHUNK))

            drain(in_copies(setB, semB, b1))
            compute_chunk(setB)
            oc = out_copies(b1)
            issue(oc)
            drain(oc)
            return carry

        lax.fori_loop(0, NCH // 2, body, 0)

    return skin


def kernel(query_xyz, query_quats, scales, opacities, sph, sk_ind, sk_w,
           node_ref_xyz, node_ref_quat, node_tgt_xyz, node_tgt_quat):
    N = query_xyz.shape[0]
    M = node_ref_xyz.shape[0]
    assert sk_ind.shape[1] == 8

    def blocks(a):
        rows, C = a.shape
        return a.T.reshape(C, rows // _B, _B).transpose(1, 0, 2).reshape(-1)

    mu_t, fr_t, s_t, o = _build(N, M)(
        query_xyz.T.reshape(-1),
        blocks(query_quats),
        scales.T.reshape(-1),
        opacities,
        blocks(sk_ind.astype(jnp.int32)),
        blocks(sk_w),
        node_ref_xyz.T.reshape(-1),
        blocks(node_ref_quat),
        node_tgt_xyz.T.reshape(-1),
        blocks(node_tgt_quat),
    )
    mu = (mu_t.reshape(N // _B, 4, _B)[:, :3, :]
          .transpose(0, 2, 1).reshape(N, 3))
    fr = (fr_t.reshape(3, N // _B, 4, _B)[:, :, :3, :]
          .transpose(1, 3, 0, 2).reshape(N, 3, 3))
    s = (s_t.reshape(N // _B, 4, _B)[:, :3, :]
         .transpose(0, 2, 1).reshape(N, 3))
    return (mu, fr, s, o, sph * jnp.float32(1.0))


# speedup vs baseline: 1.1965x; 1.0351x over previous
import functools

import jax
import jax.numpy as jnp
from jax import lax
from jax.experimental import pallas as pl
from jax.experimental.pallas import tpu as pltpu
from jax.experimental.pallas import tpu_sc as plsc

_NC = 2
_NS = 16
_NW = _NC * _NS
_L = 16
_B = 128
_CHUNK = 512
_UNROLL = 2


def _rsqrt(x):
    i = plsc.bitcast(x, jnp.int32)
    y = plsc.bitcast(jnp.int32(0x5F3759DF) - (i >> 1), jnp.float32)
    for _ in range(3):
        y = y * (1.5 - 0.5 * x * y * y)
    return y


def _inv_norm4(w, x, y, z):
    n2 = (w * w + x * x) + (y * y + z * z)
    nrm = n2 * _rsqrt(jnp.maximum(n2, 1e-30))
    return 1.0 / (nrm + 1e-8)


def _rotmat(w, x, y, z):
    n2 = (w * w + x * x) + (y * y + z * z)
    s = 2.0 / (n2 + 1e-30)
    xs, ys, zs = x * s, y * s, z * s
    wx, wy, wz = w * xs, w * ys, w * zs
    xx, xy, xz = x * xs, x * ys, x * zs
    yy, yz, zz = y * ys, y * zs, z * zs
    return ((1.0 - (yy + zz), xy - wz, xz + wy),
            (xy + wz, 1.0 - (xx + zz), yz - wx),
            (xz - wy, yz + wx, 1.0 - (xx + yy)))


@functools.lru_cache(maxsize=None)
def _build(N, M):
    assert N % (_NW * _CHUNK) == 0 and M % (_NS * _L) == 0
    assert _CHUNK % _B == 0
    G = N // _NW
    NCH = G // _CHUNK
    assert NCH % 2 == 0
    GROUPS = _CHUNK // _L
    MSL = M // _NS

    mesh = plsc.VectorSubcoreMesh(core_axis_name="c", subcore_axis_name="s")
    f32 = jnp.float32

    def in_set():
        return (pltpu.VMEM((7 * _CHUNK,), f32),
                pltpu.VMEM((4 * _CHUNK,), f32),
                pltpu.VMEM((8 * _CHUNK,), f32),
                pltpu.VMEM((8 * _CHUNK,), jnp.int32))

    @functools.partial(
        pl.kernel,
        out_type=(
            jax.ShapeDtypeStruct((4 * N,), f32),
            jax.ShapeDtypeStruct((12 * N,), f32),
            jax.ShapeDtypeStruct((4 * N,), f32),
            jax.ShapeDtypeStruct((N,), f32),
        ),
        mesh=mesh,
        compiler_params=pltpu.CompilerParams(
            needs_layout_passes=False,
            use_tc_tiling_on_sc=False,
        ),
        scratch_types=(
            pltpu.VMEM_SHARED((7 * M,), f32),
            [pltpu.VMEM((M,), f32) for _ in range(7)],
            pltpu.VMEM((14 * MSL,), f32),
            pltpu.VMEM((7 * MSL,), f32),
            in_set(),
            in_set(),
            pltpu.VMEM((21 * _CHUNK,), f32),
            pltpu.VMEM((21 * _CHUNK,), f32),
            pltpu.SemaphoreType.DMA,
            pltpu.SemaphoreType.DMA,
            pltpu.SemaphoreType.DMA,
            pltpu.SemaphoreType.DMA,
        ),
    )
    def skin(qx_h, qq_h, sc_h, op_h, ind_h, w_h,
             nrx_h, nrq_h, ntx_h, ntq_h,
             mu_h, fr_h, s_h, o_h,
             shared, tb, nin, tsl, setA, setB, cfoutA, cfoutB,
             semA, semB, osemA, osemB):
        ci = lax.axis_index("c")
        si = lax.axis_index("s")
        wid = si * _NC + ci

        iota = jnp.arange(_L, dtype=jnp.int32)

        def in_copies(bufs, sem, b):
            cfin, cqq, cw, cind = bufs
            cps = []
            for cc in range(3):
                cps.append((qx_h.at[pl.ds(cc * N + b, _CHUNK)],
                            cfin.at[pl.ds(cc * _CHUNK, _CHUNK)], sem))
                cps.append((sc_h.at[pl.ds(cc * N + b, _CHUNK)],
                            cfin.at[pl.ds((3 + cc) * _CHUNK, _CHUNK)], sem))
            cps.append((op_h.at[pl.ds(b, _CHUNK)],
                        cfin.at[pl.ds(6 * _CHUNK, _CHUNK)], sem))
            cps.append((qq_h.at[pl.ds(b * 4, 4 * _CHUNK)], cqq, sem))
            cps.append((w_h.at[pl.ds(b * 8, 8 * _CHUNK)], cw, sem))
            cps.append((ind_h.at[pl.ds(b * 8, 8 * _CHUNK)], cind, sem))
            return cps

        def issue(copies):
            for s, d, sem in copies:
                pltpu.async_copy(s, d, sem)

        def drain(copies):
            for s, d, sem in copies:
                pltpu.make_async_copy(s, d, sem).wait()


        nb = si * MSL
        handles = []
        for cc in range(3):
            handles.append(pltpu.async_copy(
                nrx_h.at[pl.ds(cc * M + nb, MSL)],
                nin.at[pl.ds(cc * MSL, MSL)], semA))
            handles.append(pltpu.async_copy(
                ntx_h.at[pl.ds(cc * M + nb, MSL)],
                nin.at[pl.ds((3 + cc) * MSL, MSL)], semA))
        handles.append(pltpu.async_copy(
            nrq_h.at[pl.ds(nb * 4, 4 * MSL)],
            nin.at[pl.ds(6 * MSL, 4 * MSL)], semA))
        handles.append(pltpu.async_copy(
            ntq_h.at[pl.ds(nb * 4, 4 * MSL)],
            nin.at[pl.ds(10 * MSL, 4 * MSL)], semA))
        issue(in_copies(setA, semA, wid * G))
        for h in handles:
            h.wait()

        def node_group(g, carry):
            def ld(row):
                return plsc.load_gather(nin, [iota + (row * MSL + g * _L)])
            blk = g // (_B // _L)
            off = (g % (_B // _L)) * _L
            qb_ = blk * (4 * _B) + off

            def ldq(qoff, cc):
                return plsc.load_gather(
                    nin, [iota + (qoff + qb_ + cc * _B)])
            rv = [ld(0), ld(1), ld(2)]
            rq = [ldq(6 * MSL, cc) for cc in range(4)]
            tv = [ld(3), ld(4), ld(5)]
            tq = [ldq(10 * MSL, cc) for cc in range(4)]
            rinv = _inv_norm4(*rq)
            tinv = _inv_norm4(*tq)
            aw, ax, ay, az = (q * tinv for q in tq)
            bw = rq[0] * rinv
            bx = -rq[1] * rinv
            by = -rq[2] * rinv
            bz = -rq[3] * rinv
            dw = aw * bw - ax * bx - ay * by - az * bz
            dx = aw * bx + ax * bw + ay * bz - az * by
            dy = aw * by - ax * bz + ay * bw + az * bx
            dz = aw * bz + ax * by - ay * bx + az * bw
            R = _rotmat(dw, dx, dy, dz)
            t = [tv[r] - (R[r][0] * rv[0] + R[r][1] * rv[1] + R[r][2] * rv[2])
                 for r in range(3)]
            base = g * _L
            for row, val in enumerate((dw, dx, dy, dz, t[0], t[1], t[2])):
                plsc.store_scatter(tsl, [iota + (row * MSL + base)], val)
            return carry

        lax.fori_loop(0, MSL // _L, node_group, 0)
        for comp in range(7):
            pltpu.sync_copy(tsl.at[pl.ds(comp * MSL, MSL)],
                            shared.at[pl.ds(comp * M + si * MSL, MSL)])
        plsc.subcore_barrier()
        for comp in range(7):
            pltpu.sync_copy(shared.at[pl.ds(comp * M, M)], tb[comp])

        g0 = wid * G

        def do_group(bufs, cfout, g):
            cfin, cqq, cw, cind = bufs
            base = g * _L
            blk = g // (_B // _L)
            off = (g % (_B // _L)) * _L
            qbase = blk * (4 * _B) + off
            wbase = blk * (8 * _B) + off

            ks = [cind[pl.ds(wbase + k * _B, _L)] for k in range(8)]
            ws = [cw[pl.ds(wbase + k * _B, _L)] for k in range(8)]
            wsum = (((ws[0] + ws[1]) + (ws[2] + ws[3]))
                    + ((ws[4] + ws[5]) + (ws[6] + ws[7])))
            winv = 1.0 / (wsum + 1e-8)

            q0 = [plsc.load_gather(tb[cc], [ks[0]]) for cc in range(4)]
            wk = [ws[0]]
            for k in range(1, 8):
                qk = [plsc.load_gather(tb[cc], [ks[k]]) for cc in range(4)]
                d = ((q0[0] * qk[0] + q0[1] * qk[1])
                     + (q0[2] * qk[2] + q0[3] * qk[3]))
                wk.append(jnp.where(d < 0, -ws[k], ws[k]))
                if k == 1:
                    aq = [wk[1] * q for q in qk]
                else:
                    aq = [aq[cc] + wk[k] * qk[cc] for cc in range(4)]
            aq = [aq[cc] + wk[0] * q0[cc] for cc in range(4)]
            at = None
            for k in range(8):
                tk = [plsc.load_gather(tb[4 + cc], [ks[k]])
                      for cc in range(3)]
                if at is None:
                    at = [ws[0] * t for t in tk]
                else:
                    at = [at[cc] + ws[k] * tk[cc] for cc in range(3)]

            tb_ = [a * winv for a in at]
            Rb = _rotmat(aq[0], aq[1], aq[2], aq[3])

            def ldf(row):
                return cfin[pl.ds(row * _CHUNK + base, _L)]

            mub = blk * (4 * _B) + off
            v = [ldf(0), ldf(1), ldf(2)]
            for r in range(3):
                cfout[pl.ds(mub + r * _B, _L)] = (
                    (Rb[r][0] * v[0] + Rb[r][1] * v[1])
                    + (Rb[r][2] * v[2] + tb_[r]))

            qr = [cqq[pl.ds(qbase + cc * _B, _L)] for cc in range(4)]
            mw = aq[0] * qr[0] - aq[1] * qr[1] - aq[2] * qr[2] - aq[3] * qr[3]
            mx = aq[0] * qr[1] + aq[1] * qr[0] + aq[2] * qr[3] - aq[3] * qr[2]
            my = aq[0] * qr[2] - aq[1] * qr[3] + aq[2] * qr[0] + aq[3] * qr[1]
            mz = aq[0] * qr[3] + aq[1] * qr[2] - aq[2] * qr[1] + aq[3] * qr[0]
            Rm = _rotmat(mw, mx, my, mz)
            for r in range(3):
                frb = (4 + 4 * r) * _CHUNK + mub
                for col in range(3):
                    cfout[pl.ds(frb + col * _B, _L)] = Rm[r][col]

            sb = 16 * _CHUNK + mub
            for cc in range(3):
                cfout[pl.ds(sb + cc * _B, _L)] = jnp.exp(ldf(3 + cc))
            cfout[pl.ds(20 * _CHUNK + base, _L)] = (
                1.0 / (1.0 + jnp.exp(-ldf(6))))

        def compute_chunk(bufs, cfout):
            def group_fn(j, carry2):
                for u in range(_UNROLL):
                    do_group(bufs, cfout, j * _UNROLL + u)
                return carry2
            lax.fori_loop(0, GROUPS // _UNROLL, group_fn, 0)

        def out_copies(cfout, osem, b):
            cps = [(cfout.at[pl.ds(0, 4 * _CHUNK)],
                    mu_h.at[pl.ds(b * 4, 4 * _CHUNK)], osem)]
            for r in range(3):
                cps.append((cfout.at[pl.ds((4 + 4 * r) * _CHUNK, 4 * _CHUNK)],
                            fr_h.at[pl.ds(r * (4 * N) + b * 4, 4 * _CHUNK)],
                            osem))
            cps.append((cfout.at[pl.ds(16 * _CHUNK, 4 * _CHUNK)],
                        s_h.at[pl.ds(b * 4, 4 * _CHUNK)], osem))
            cps.append((cfout.at[pl.ds(20 * _CHUNK, _CHUNK)],
                        o_h.at[pl.ds(b, _CHUNK)], osem))
            return cps

        def body(i, carry):
            b0 = g0 + (2 * i) * _CHUNK
            b1 = b0 + _CHUNK
            issue(in_copies(setB, semB, b1))
            drain(in_copies(setA, semA, b0))

            @pl.when(i > 0)
            def _drainA():
                drain(out_copies(cfoutA, osemA, b0))

            compute_chunk(setA, cfoutA)
            issue(out_copies(cfoutA, osemA, b0))

            @pl.when(i + 1 < NCH // 2)
            def _prefetch():
                issue(in_copies(setA, semA, b1 + _CHUNK))

            drain(in_copies(setB, semB, b1))

            @pl.when(i > 0)
            def _drainB():
                drain(out_copies(cfoutB, osemB, b1))

            compute_chunk(setB, cfoutB)
            issue(out_copies(cfoutB, osemB, b1))
            return carry

        lax.fori_loop(0, NCH // 2, body, 0)
        drain(out_copies(cfoutA, osemA, g0))
        drain(out_copies(cfoutB, osemB, g0))

    return skin


def kernel(query_xyz, query_quats, scales, opacities, sph, sk_ind, sk_w,
           node_ref_xyz, node_ref_quat, node_tgt_xyz, node_tgt_quat):
    N = query_xyz.shape[0]
    M = node_ref_xyz.shape[0]
    assert sk_ind.shape[1] == 8

    def blocks(a):
        rows, C = a.shape
        return a.T.reshape(C, rows // _B, _B).transpose(1, 0, 2).reshape(-1)

    mu_t, fr_t, s_t, o = _build(N, M)(
        query_xyz.T.reshape(-1),
        blocks(query_quats),
        scales.T.reshape(-1),
        opacities,
        blocks(sk_ind.astype(jnp.int32)),
        blocks(sk_w),
        node_ref_xyz.T.reshape(-1),
        blocks(node_ref_quat),
        node_tgt_xyz.T.reshape(-1),
        blocks(node_tgt_quat),
    )
    mu = (mu_t.reshape(N // _B, 4, _B)[:, :3, :]
          .transpose(0, 2, 1).reshape(N, 3))
    fr = (fr_t.reshape(3, N // _B, 4, _B)[:, :, :3, :]
          .transpose(1, 3, 0, 2).reshape(N, 3, 3))
    s = (s_t.reshape(N // _B, 4, _B)[:, :3, :]
         .transpose(0, 2, 1).reshape(N, 3))
    return (mu, fr, s, o, sph * jnp.float32(1.0))
